# Initial kernel scaffold; baseline (speedup 1.0000x reference)
import jax
import jax.numpy as jnp
from jax.experimental import pallas as pl


def kernel(x, lut):
    raise NotImplementedError("write your pallas kernel here")


# speedup vs baseline: 2.1088x; 2.1088x over previous
import math

import jax
import jax.numpy as jnp
from jax import lax
from jax.experimental import pallas as pl
from jax.experimental.pallas import tpu as pltpu
from jax.experimental.pallas import tpu_sc as plsc

_D = 158
_E = 12
_L = 16
_NC = 2
_NS = 16
_NW = _NC * _NS
_SCALE = math.sqrt(_D)

_B1, _B2 = 16384, 200
_N = _B1 * _B2
_NT = _N // _NW
_C = 256
_NCH = _NT // _C
_DU = 16


def _expand_chunk(lut_v, idx_ref, out_ref):
    iota = lax.iota(jnp.int32, _L)

    @plsc.parallel_loop(0, _C // _L)
    def gbody(g):
        tok = idx_ref[pl.ds(g * _L, _L)]
        rowbase = (iota + g * _L) * _D

        @plsc.parallel_loop(0, _D, unroll=_DU)
        def dloop(d):
            val = plsc.load_gather(lut_v, [tok + d * _L])
            plsc.store_scatter(out_ref, [rowbase + d], val)


def _body(x_hbm, lut_hbm, out_hbm, lut_v, idx0, idx1, o0, o1,
          s_lut, s_in0, s_in1, s_out0, s_out1):
    wid = lax.axis_index("s") * _NC + lax.axis_index("c")
    base = wid * _NT

    cp_lut = pltpu.async_copy(lut_hbm, lut_v, s_lut)
    cp0 = pltpu.async_copy(x_hbm.at[pl.ds(base, _C)], idx0, s_in0)
    cp1 = pltpu.async_copy(x_hbm.at[pl.ds(base + _C, _C)], idx1, s_in1)
    cp_lut.wait()

    def sbody(i, carry):
        lut_v[pl.ds(i * _L, _L)] = lut_v[pl.ds(i * _L, _L)] * _SCALE
        return carry

    lax.fori_loop(0, _D, sbody, 0)

    cp0.wait()
    _expand_chunk(lut_v, idx0, o0)
    pltpu.async_copy(o0, out_hbm.at[pl.ds(base * _D, _C * _D)], s_out0)
    pltpu.async_copy(x_hbm.at[pl.ds(base + 2 * _C, _C)], idx0, s_in0)

    cp1.wait()
    _expand_chunk(lut_v, idx1, o1)
    pltpu.async_copy(o1, out_hbm.at[pl.ds((base + _C) * _D, _C * _D)], s_out1)
    pltpu.async_copy(x_hbm.at[pl.ds(base + 3 * _C, _C)], idx1, s_in1)

    def pair(p, carry):
        c = 2 * p
        for b, (idx_v, out_v, s_in, s_out) in enumerate(
                ((idx0, o0, s_in0, s_out0), (idx1, o1, s_in1, s_out1))):
            tok = base + (c + b) * _C
            pltpu.make_async_copy(
                out_v, out_hbm.at[pl.ds(0, _C * _D)], s_out).wait()
            pltpu.make_async_copy(
                x_hbm.at[pl.ds(0, _C)], idx_v, s_in).wait()
            _expand_chunk(lut_v, idx_v, out_v)
            pltpu.async_copy(out_v, out_hbm.at[pl.ds(tok * _D, _C * _D)], s_out)
            nxt = jnp.minimum(tok + 2 * _C, _N - _C)
            pltpu.async_copy(x_hbm.at[pl.ds(nxt, _C)], idx_v, s_in)
        return carry

    lax.fori_loop(1, _NCH // 2, pair, 0)

    pltpu.make_async_copy(o0, out_hbm.at[pl.ds(0, _C * _D)], s_out0).wait()
    pltpu.make_async_copy(o1, out_hbm.at[pl.ds(0, _C * _D)], s_out1).wait()
    pltpu.make_async_copy(x_hbm.at[pl.ds(0, _C)], idx0, s_in0).wait()
    pltpu.make_async_copy(x_hbm.at[pl.ds(0, _C)], idx1, s_in1).wait()


def kernel(x, lut):
    x_flat = x.reshape(-1).astype(jnp.int32)
    lut_t = jnp.zeros((_D, _L), jnp.float32).at[:, :_E].set(
        lut.astype(jnp.float32).T).reshape(-1)

    run = pl.kernel(
        _body,
        out_type=jax.ShapeDtypeStruct((_N * _D,), jnp.float32),
        mesh=plsc.VectorSubcoreMesh(
            core_axis_name="c", subcore_axis_name="s",
            num_cores=_NC, num_subcores=_NS),
        compiler_params=pltpu.CompilerParams(needs_layout_passes=False),
        scratch_types=[
            pltpu.VMEM((_D * _L,), jnp.float32),
            pltpu.VMEM((_C,), jnp.int32),
            pltpu.VMEM((_C,), jnp.int32),
            pltpu.VMEM((_C * _D,), jnp.float32),
            pltpu.VMEM((_C * _D,), jnp.float32),
            pltpu.SemaphoreType.DMA,
            pltpu.SemaphoreType.DMA,
            pltpu.SemaphoreType.DMA,
            pltpu.SemaphoreType.DMA,
            pltpu.SemaphoreType.DMA,
        ],
    )
    out_flat = run(x_flat, lut_t)
    return out_flat.reshape(_B1, _B2, _D)
